# Initial kernel scaffold; baseline (speedup 1.0000x reference)
import jax
import jax.numpy as jnp
from jax.experimental import pallas as pl


def kernel(input1, input2, eps, iters):
    raise NotImplementedError("write your pallas kernel here")


# speedup vs baseline: 32.8086x; 32.8086x over previous
import jax
import jax.numpy as jnp
from jax.experimental import pallas as pl
from jax.experimental.pallas import tpu as pltpu

_NEG_INF = float("-inf")
_POS_INF = float("inf")


def _auction_body(d2_ref, eps_ref, it_ref, dist_ref, asg_ref):
    n = d2_ref.shape[1]
    m = d2_ref.shape[2]
    d2 = d2_ref[0]

    eps = eps_ref[0, 0]
    num_iters = it_ref[0, 0]

    col = jax.lax.broadcasted_iota(jnp.int32, (n, m), 1)
    row = jax.lax.broadcasted_iota(jnp.int32, (n, m), 0)

    def body(_, state):
        asg, inv, price = state
        value = d2 + price
        v1 = jnp.min(value, axis=1, keepdims=True)
        bj = jnp.min(jnp.where(value == v1, col, jnp.int32(m)),
                     axis=1, keepdims=True)
        masked = jnp.where(col == bj, _POS_INF, value)
        v2 = jnp.min(masked, axis=1, keepdims=True)
        incr = (v2 - v1) + eps
        unas = asg < 0
        bid_mask = unas & (col == bj)
        bids = jnp.where(bid_mask, incr, _NEG_INF)
        max_incr = jnp.max(bids, axis=0, keepdims=True)
        has_bid = max_incr > _NEG_INF
        winner = jnp.min(
            jnp.where(bid_mask & (bids == max_incr), row, jnp.int32(n)),
            axis=0, keepdims=True)
        new_price = jnp.where(has_bid, price + max_incr, price)
        evict = jnp.any((inv == row) & has_bid, axis=1, keepdims=True)
        won = jnp.any(bid_mask & (winner == row), axis=1, keepdims=True)
        new_asg = jnp.where(won, bj, jnp.where(evict, jnp.int32(-1), asg))
        new_inv = jnp.where(has_bid, winner, inv)
        return new_asg, new_inv, new_price

    init = (jnp.full((n, 1), -1, jnp.int32),
            jnp.full((1, m), -1, jnp.int32),
            jnp.zeros((1, m), jnp.float32))
    asg, _, _ = jax.lax.fori_loop(0, num_iters, body, init)

    a = jnp.clip(asg, 0, m - 1)
    dist = jnp.sum(jnp.where(col == a, d2, 0.0), axis=1, keepdims=True)
    dist = jnp.where(asg >= 0, dist, 0.0)
    dist_ref[0] = dist
    asg_ref[0] = asg


def kernel(input1, input2, eps, iters):
    B, n, _ = input1.shape
    m = input2.shape[1]
    s1 = jnp.sum(input1 * input1, axis=-1)[:, :, None]
    s2 = jnp.sum(input2 * input2, axis=-1)[:, None, :]
    cross = jnp.einsum('bnd,bmd->bnm', input1, input2)
    d2 = jnp.maximum(s1 + s2 - 2.0 * cross, 0.0)

    eps_arr = jnp.asarray(eps, jnp.float32).reshape(1, 1)
    it_arr = jnp.asarray(iters, jnp.int32).reshape(1, 1)
    dist3, asg3 = pl.pallas_call(
        _auction_body,
        grid=(B,),
        in_specs=[
            pl.BlockSpec((1, n, m), lambda b: (b, 0, 0)),
            pl.BlockSpec(memory_space=pltpu.SMEM),
            pl.BlockSpec(memory_space=pltpu.SMEM),
        ],
        out_specs=[
            pl.BlockSpec((1, n, 1), lambda b: (b, 0, 0)),
            pl.BlockSpec((1, n, 1), lambda b: (b, 0, 0)),
        ],
        out_shape=[
            jax.ShapeDtypeStruct((B, n, 1), jnp.float32),
            jax.ShapeDtypeStruct((B, n, 1), jnp.int32),
        ],
    )(d2, eps_arr, it_arr)
    return dist3[..., 0], asg3[..., 0]
